# Initial kernel scaffold; baseline (speedup 1.0000x reference)
import jax
import jax.numpy as jnp
from jax.experimental import pallas as pl


def kernel(vocab_id_list, table):
    raise NotImplementedError("write your pallas kernel here")


# speedup vs baseline: 2.9719x; 2.9719x over previous
import functools

import jax
import jax.numpy as jnp
from jax import lax
from jax.experimental import pallas as pl
from jax.experimental.pallas import tpu as pltpu
from jax.experimental.pallas import tpu_sc as plsc

VOCAB = 100000
EMBED_DIM = 128
BATCH = 4096
HIST = 50
B = BATCH * HIST
NUM_WORKERS = 32
ROWS_PER_W = B // NUM_WORKERS
CHUNK = 128
NCHUNK = ROWS_PER_W // CHUNK


def _make_gather():
    mesh = plsc.VectorSubcoreMesh(core_axis_name="c", subcore_axis_name="s")

    @functools.partial(
        pl.kernel,
        mesh=mesh,
        out_type=jax.ShapeDtypeStruct((B, EMBED_DIM), jnp.float32),
        scratch_types=[
            pltpu.VMEM((ROWS_PER_W,), jnp.int32),
            pltpu.VMEM((CHUNK, EMBED_DIM), jnp.float32),
            pltpu.SemaphoreType.DMA,
            pltpu.SemaphoreType.DMA,
        ],
    )
    def gather_kernel(idx_hbm, table_hbm, out_hbm, idx_v, buf, gsem, psem):
        wid = lax.axis_index("s") * 2 + lax.axis_index("c")
        base = wid * ROWS_PER_W
        pltpu.sync_copy(idx_hbm.at[pl.ds(base, ROWS_PER_W)], idx_v)

        def body(j, carry):
            pltpu.async_copy(
                table_hbm.at[idx_v.at[pl.ds(j * CHUNK, CHUNK)]], buf, gsem
            ).wait()
            pltpu.async_copy(
                buf, out_hbm.at[pl.ds(base + j * CHUNK, CHUNK)], psem
            ).wait()
            return carry

        lax.fori_loop(0, NCHUNK, body, 0)

    return gather_kernel


_gather = _make_gather()


def kernel(vocab_id_list, table):
    idx = vocab_id_list.reshape(B)
    out = _gather(idx, table)
    return out.reshape(BATCH, HIST, EMBED_DIM)


# speedup vs baseline: 3.3187x; 1.1167x over previous
import functools

import jax
import jax.numpy as jnp
from jax import lax
from jax.experimental import pallas as pl
from jax.experimental.pallas import tpu as pltpu
from jax.experimental.pallas import tpu_sc as plsc

VOCAB = 100000
EMBED_DIM = 128
BATCH = 4096
HIST = 50
B = BATCH * HIST
NUM_WORKERS = 32
ROWS_PER_W = B // NUM_WORKERS
CHUNK = 128
NCHUNK = ROWS_PER_W // CHUNK
NBUF = 5
GROUPS = NCHUNK // NBUF


def _make_gather():
    mesh = plsc.VectorSubcoreMesh(core_axis_name="c", subcore_axis_name="s")

    scratch = [pltpu.VMEM((ROWS_PER_W,), jnp.int32)]
    scratch += [pltpu.VMEM((CHUNK, EMBED_DIM), jnp.float32) for _ in range(NBUF)]
    scratch += [pltpu.SemaphoreType.DMA for _ in range(2 * NBUF)]

    @functools.partial(
        pl.kernel,
        mesh=mesh,
        out_type=jax.ShapeDtypeStruct((B, EMBED_DIM), jnp.float32),
        scratch_types=scratch,
    )
    def gather_kernel(idx_hbm, table_hbm, out_hbm, idx_v, *bufs_and_sems):
        bufs = bufs_and_sems[:NBUF]
        gsem = bufs_and_sems[NBUF:2 * NBUF]
        psem = bufs_and_sems[2 * NBUF:]
        wid = lax.axis_index("s") * 2 + lax.axis_index("c")
        base = wid * ROWS_PER_W
        pltpu.sync_copy(idx_hbm.at[pl.ds(base, ROWS_PER_W)], idx_v)

        def fire_gather(j, s):
            pltpu.async_copy(
                table_hbm.at[idx_v.at[pl.ds(j * CHUNK, CHUNK)]], bufs[s], gsem[s]
            )

        def fire_put(j, s):
            pltpu.async_copy(
                bufs[s], out_hbm.at[pl.ds(base + j * CHUNK, CHUNK)], psem[s]
            )

        for s in range(NBUF):
            fire_gather(s, s)

        def body(g, carry):
            for s in range(NBUF):
                j = g * NBUF + s
                pltpu.make_async_copy(
                    table_hbm.at[idx_v.at[pl.ds(j * CHUNK, CHUNK)]],
                    bufs[s], gsem[s],
                ).wait()
                fire_put(j, s)
            for s in range(NBUF):
                j = g * NBUF + s
                pltpu.make_async_copy(
                    bufs[s], out_hbm.at[pl.ds(base + j * CHUNK, CHUNK)], psem[s]
                ).wait()
                fire_gather((g + 1) * NBUF + s, s)
            return carry

        lax.fori_loop(0, GROUPS - 1, body, 0)

        g = GROUPS - 1
        for s in range(NBUF):
            j = g * NBUF + s
            pltpu.make_async_copy(
                table_hbm.at[idx_v.at[pl.ds(j * CHUNK, CHUNK)]], bufs[s], gsem[s]
            ).wait()
            fire_put(j, s)
        for s in range(NBUF):
            j = g * NBUF + s
            pltpu.make_async_copy(
                bufs[s], out_hbm.at[pl.ds(base + j * CHUNK, CHUNK)], psem[s]
            ).wait()

    return gather_kernel


_gather = _make_gather()


def kernel(vocab_id_list, table):
    idx = vocab_id_list.reshape(B)
    out = _gather(idx, table)
    return out.reshape(BATCH, HIST, EMBED_DIM)
